# baseline (speedup 1.0000x reference)
import functools

import jax
import jax.numpy as jnp
from jax import lax
from jax.experimental import pallas as pl
from jax.experimental.pallas import tpu as pltpu
from jax.experimental.pallas import tpu_sc as plsc

F32 = jnp.float32
BF16 = jnp.bfloat16

N0 = 2048
D = 128
KK1, P1 = 1843, 1920
KK2, P2 = 1290, 1536


def _gcn(A, h_in, Wt, b, *, deg=None, scale=None, resid=None, resid2=None,
         pool=None, score_kk=0, mask_rows=0, name="gcn"):
    rows = A.shape[0]
    has_deg = deg is not None
    has_scale = scale is not None
    has_resid = resid is not None
    has_resid2 = resid2 is not None
    has_pool = pool is not None

    def body(*refs):
        it = iter(refs)
        a_ref = next(it)
        h_ref = next(it)
        wt_ref = next(it)
        b_ref = next(it)
        deg_ref = next(it) if has_deg else None
        sc_ref = next(it) if has_scale else None
        r_ref = next(it) if has_resid else None
        r2_ref = next(it) if has_resid2 else None
        p_ref = next(it) if has_pool else None
        pb_ref = next(it) if has_pool else None
        o_ref = next(it)
        o2_ref = next(it) if has_resid2 else None
        s_ref = next(it) if has_pool else None

        hv = h_ref[...]
        if has_scale:
            hv = hv * sc_ref[...]
        t = jnp.dot(a_ref[...], hv, preferred_element_type=F32)
        if has_deg:
            t = t / deg_ref[...]
        out = jnp.maximum(jnp.dot(t, wt_ref[...], preferred_element_type=F32)
                          + b_ref[...], 0.0)
        if has_resid:
            out = out + r_ref[...]
        if mask_rows:
            ri = lax.broadcasted_iota(jnp.int32, out.shape, 0)
            out = jnp.where(ri < mask_rows, out, 0.0)
        o_ref[...] = out
        if has_resid2:
            o2_ref[...] = out + r2_ref[...]
        if has_pool:
            s = jnp.sum(out * p_ref[...], axis=1, keepdims=True) + pb_ref[...]
            s = jax.nn.sigmoid(s)
            if score_kk:
                ri1 = lax.broadcasted_iota(jnp.int32, s.shape, 0)
                s = jnp.where(ri1 < score_kk, s, -1.0)
            s_ref[...] = s

    inputs = [A, h_in, Wt, b]
    if has_deg:
        inputs.append(deg)
    if has_scale:
        inputs.append(scale)
    if has_resid:
        inputs.append(resid)
    if has_resid2:
        inputs.append(resid2)
    if has_pool:
        inputs.extend(pool)
    out_shape = [jax.ShapeDtypeStruct((rows, D), F32)]
    if has_resid2:
        out_shape.append(jax.ShapeDtypeStruct((rows, D), F32))
    if has_pool:
        out_shape.append(jax.ShapeDtypeStruct((rows, 1), F32))
    res = pl.pallas_call(body, out_shape=out_shape, name=name)(*inputs)
    return res[0] if len(res) == 1 else tuple(res)


def _u16_pass(g):
    BLK = 256
    nb = N0 // BLK

    def body(g_ref, u_ref, ut_ref):
        u = (g_ref[...] != 0).astype(BF16)
        u_ref[...] = u
        ut_ref[...] = u.T

    return pl.pallas_call(
        body,
        grid=(nb, nb),
        in_specs=[pl.BlockSpec((BLK, BLK), lambda i, j: (i, j))],
        out_specs=[pl.BlockSpec((BLK, BLK), lambda i, j: (i, j)),
                   pl.BlockSpec((BLK, BLK), lambda i, j: (j, i))],
        out_shape=[jax.ShapeDtypeStruct((N0, N0), BF16),
                   jax.ShapeDtypeStruct((N0, N0), BF16)],
        name="u16_pass",
    )(g)


def _adj_pool(Crows, Drows, kk, *, pc=0, name="adj_pool"):
    P, M = Crows.shape
    BLK = 128
    nb = P // BLK
    steps = (pc // BLK) if pc else nb

    def body(c_ref, d_ref, b_ref, deg_ref, *rest):
        i = pl.program_id(0)
        icl = jnp.minimum(i, nb - 1)
        cnt = lax.dot_general(c_ref[...], d_ref[...], (((1,), (1,)), ((), ())),
                              preferred_element_type=F32)
        ci = lax.broadcasted_iota(jnp.int32, (BLK, P), 1)
        ri_cl = icl * BLK + lax.broadcasted_iota(jnp.int32, (BLK, P), 0)
        keep = (cnt != 0) & (ri_cl < kk) & (ci < kk)
        Bv = jnp.where(keep, 1.0, 0.0)
        b_ref[...] = Bv
        rs = jnp.sum(Bv, axis=1, keepdims=True)
        ri1 = icl * BLK + lax.broadcasted_iota(jnp.int32, (BLK, 1), 0)
        deg_ref[...] = jnp.where(ri1 < kk, rs, 1.0)
        if pc:
            ri_act = i * BLK + lax.broadcasted_iota(jnp.int32, (BLK, P), 0)
            b16 = jnp.where(keep & (ri_act < kk), 1.0, 0.0).astype(BF16)
            b16f = jnp.concatenate(
                [b16, jnp.zeros((BLK, pc - P), BF16)], axis=1)
            rest[0][...] = b16f
            rest[1][...] = b16f.T

    out_specs = [pl.BlockSpec((BLK, P), lambda i: (jnp.minimum(i, nb - 1), 0)),
                 pl.BlockSpec((BLK, 1), lambda i: (jnp.minimum(i, nb - 1), 0))]
    out_shape = [jax.ShapeDtypeStruct((P, P), F32),
                 jax.ShapeDtypeStruct((P, 1), F32)]
    if pc:
        out_specs += [pl.BlockSpec((BLK, pc), lambda i: (i, 0)),
                      pl.BlockSpec((pc, BLK), lambda i: (0, i))]
        out_shape += [jax.ShapeDtypeStruct((pc, pc), BF16),
                      jax.ShapeDtypeStruct((pc, pc), BF16)]

    return pl.pallas_call(
        body,
        grid=(steps,),
        in_specs=[pl.BlockSpec((BLK, M),
                               lambda i: (jnp.minimum(i, nb - 1), 0)),
                  pl.BlockSpec((P, M), lambda i: (0, 0))],
        out_specs=out_specs,
        out_shape=out_shape,
        name=name,
    )(Crows, Drows)


def _rank(s_col, s_row, name="rank"):
    P = s_col.shape[0]
    BLK = 128

    def body(sc_ref, sr_ref, o_ref):
        i = pl.program_id(0)
        sc = sc_ref[...]
        sr = sr_ref[...]
        ri = i * BLK + lax.broadcasted_iota(jnp.int32, (BLK, P), 0)
        ci = lax.broadcasted_iota(jnp.int32, (BLK, P), 1)
        before = (sr > sc) | ((sr == sc) & (ci < ri))
        o_ref[...] = jnp.sum(before.astype(F32), axis=1, keepdims=True)

    return pl.pallas_call(
        body,
        grid=(P // BLK,),
        in_specs=[pl.BlockSpec((BLK, 1), lambda i: (i, 0)),
                  pl.BlockSpec((1, P), lambda i: (0, 0))],
        out_specs=pl.BlockSpec((BLK, 1), lambda i: (i, 0)),
        out_shape=jax.ShapeDtypeStruct((P, 1), F32),
        name=name,
    )(s_col, s_row)


def _select(rank_row, s_row, name="select"):
    P = rank_row.shape[1]
    BLK = 128

    def body(r_ref, s_ref, i_ref, v_ref):
        i = pl.program_id(0)
        rr = r_ref[...]
        sr = s_ref[...]
        rg = (i * BLK + lax.broadcasted_iota(jnp.int32, (BLK, P), 0)).astype(F32)
        match = (rr == rg).astype(F32)
        ci = lax.broadcasted_iota(jnp.int32, (BLK, P), 1).astype(F32)
        i_ref[...] = jnp.sum(match * ci, axis=1, keepdims=True).astype(jnp.int32)
        v_ref[...] = jnp.sum(match * sr, axis=1, keepdims=True)

    return pl.pallas_call(
        body,
        grid=(P // BLK,),
        in_specs=[pl.BlockSpec((1, P), lambda i: (0, 0)),
                  pl.BlockSpec((1, P), lambda i: (0, 0))],
        out_specs=[pl.BlockSpec((BLK, 1), lambda i: (i, 0)),
                   pl.BlockSpec((BLK, 1), lambda i: (i, 0))],
        out_shape=[jax.ShapeDtypeStruct((P, 1), jnp.int32),
                   jax.ShapeDtypeStruct((P, 1), F32)],
        name=name,
    )(rank_row, s_row)


def _gather_multi(tables, idx, B, CH):
    info = plsc.get_sparse_core_info()
    NW = info.num_cores * info.num_subcores
    nw_active = NW
    b_per_w = B // nw_active
    while b_per_w % 8 or B % nw_active:
        nw_active -= 1
        b_per_w = B // nw_active
    n_ch = b_per_w // CH
    mesh = plsc.VectorSubcoreMesh(core_axis_name="c", subcore_axis_name="s")

    scratch = [pltpu.VMEM((CH,), jnp.int32)]
    for t in tables:
        scratch.append(pltpu.VMEM((CH, t.shape[1]), t.dtype))
    scratch.append(pltpu.SemaphoreType.DMA)

    @functools.partial(
        pl.kernel, mesh=mesh,
        out_type=[jax.ShapeDtypeStruct((B, t.shape[1]), t.dtype)
                  for t in tables],
        scratch_types=scratch,
    )
    def k(*refs):
        nt = len(tables)
        tab = refs[:nt]
        idx_hbm = refs[nt]
        outs = refs[nt + 1:2 * nt + 1]
        idx_v = refs[2 * nt + 1]
        bufs = refs[2 * nt + 2:3 * nt + 2]
        sem = refs[3 * nt + 2]
        wid = lax.axis_index("s") * info.num_cores + lax.axis_index("c")

        @pl.when(wid < nw_active)
        def _():
            base = wid * b_per_w
            for c in range(n_ch):
                off = base + c * CH
                pltpu.sync_copy(idx_hbm.at[pl.ds(off, CH)], idx_v)
                handles = [pltpu.async_copy(tab[t].at[idx_v], bufs[t], sem)
                           for t in range(nt)]
                for hdl in handles:
                    hdl.wait()
                for t in range(nt):
                    pltpu.sync_copy(bufs[t], outs[t].at[pl.ds(off, CH)])

    return k(*tables, idx)


def _pack_bf16(x):
    V, Dw = x.shape
    return lax.bitcast_convert_type(x.reshape(V, Dw // 2, 2), jnp.int32)


def _unpack_bf16(y):
    B, W = y.shape
    return lax.bitcast_convert_type(y, BF16).reshape(B, 2 * W)


def kernel(g, h, params):
    p = params
    W0t = p["down_W"][0].T
    W1t = p["down_W"][1].T
    Wbt = p["bottom_W"].T
    Wu0t = p["up_W"][0].T
    Wu1t = p["up_W"][1].T
    b0 = p["down_b"][0].reshape(1, D)
    b1 = p["down_b"][1].reshape(1, D)
    bb = p["bottom_b"].reshape(1, D)
    bu0 = p["up_b"][0].reshape(1, D)
    bu1 = p["up_b"][1].reshape(1, D)
    p0 = p["pool_W"][0].reshape(1, D)
    p1 = p["pool_W"][1].reshape(1, D)
    pb0 = p["pool_b"][0].reshape(1, 1)
    pb1 = p["pool_b"][1].reshape(1, 1)

    h1, s1 = _gcn(g, h, W0t, b0, pool=(p0, pb0), name="gcn_down0")
    rank1 = _rank(s1, s1.reshape(1, N0), name="rank1")
    idxs1, vals1 = _select(rank1.reshape(1, N0), s1.reshape(1, N0),
                           name="select1")
    idx1f = idxs1.reshape(N0)
    rank1i = rank1.astype(jnp.int32).reshape(N0)

    U16, U16T = _u16_pass(g)
    U_rp, UT_rp, h1_r = _gather_multi(
        [_pack_bf16(U16), _pack_bf16(U16T), h1], idx1f, P1, 32)
    U_r, UT_r = _unpack_bf16(U_rp), _unpack_bf16(UT_rp)

    B1, deg1, B116, B116T = _adj_pool(U_r, UT_r, KK1, pc=N0,
                                      name="adj_pool1")

    h2, s2 = _gcn(B1, h1_r, W1t, b1, deg=deg1, scale=vals1[:P1],
                  pool=(p1, pb1), score_kk=KK1, name="gcn_down1")
    rank2 = _rank(s2, s2.reshape(1, P1), name="rank2")
    idxs2, vals2 = _select(rank2.reshape(1, P1), s2.reshape(1, P1),
                           name="select2")
    idx2f = idxs2.reshape(P1)
    rank2i = rank2.astype(jnp.int32).reshape(P1)

    B1_rp, B1T_rp, h2_r = _gather_multi(
        [_pack_bf16(B116), _pack_bf16(B116T), h2], idx2f, P2, 48)
    B1_r, B1T_r = _unpack_bf16(B1_rp), _unpack_bf16(B1T_rp)

    B2, deg2 = _adj_pool(B1_r, B1T_r, KK2, name="adj_pool2")

    hb = _gcn(B2, h2_r, Wbt, bb, deg=deg2, scale=vals2[:P2], mask_rows=KK2,
              name="gcn_bottom")

    hb_pad = jnp.pad(hb, ((0, P1 - P2), (0, 0)))
    (nh_b,) = _gather_multi([hb_pad], rank2i, P1, 64)
    h_u0 = _gcn(B1, nh_b, Wu0t, bu0, deg=deg1, resid=h2, mask_rows=KK1,
                name="gcn_up0")

    h_u0_pad = jnp.pad(h_u0, ((0, N0 - P1), (0, 0)))
    (nh,) = _gather_multi([h_u0_pad], rank1i, N0, 64)
    h_u1, h_fin = _gcn(g, nh, Wu1t, bu1, resid=h1, resid2=h, name="gcn_up1")

    return (h_u0[:KK1], h_u1, h_fin)


# speedup vs baseline: 3.0702x; 3.0702x over previous
import functools

import jax
import jax.numpy as jnp
from jax import lax
from jax.experimental import pallas as pl
from jax.experimental.pallas import tpu as pltpu
from jax.experimental.pallas import tpu_sc as plsc

F32 = jnp.float32
BF16 = jnp.bfloat16

N0 = 2048
D = 128
KK1, P1 = 1843, 1920
KK2, P2 = 1290, 1536


def _gcn(A, h_in, Wt, b, *, deg=None, scale=None, resid=None, resid2=None,
         pool=None, score_kk=0, mask_rows=0, name="gcn"):
    rows = A.shape[0]
    has_deg = deg is not None
    has_scale = scale is not None
    has_resid = resid is not None
    has_resid2 = resid2 is not None
    has_pool = pool is not None

    def body(*refs):
        it = iter(refs)
        a_ref = next(it)
        h_ref = next(it)
        wt_ref = next(it)
        b_ref = next(it)
        deg_ref = next(it) if has_deg else None
        sc_ref = next(it) if has_scale else None
        r_ref = next(it) if has_resid else None
        r2_ref = next(it) if has_resid2 else None
        p_ref = next(it) if has_pool else None
        pb_ref = next(it) if has_pool else None
        o_ref = next(it)
        o2_ref = next(it) if has_resid2 else None
        s_ref = next(it) if has_pool else None

        hv = h_ref[...]
        if has_scale:
            hv = hv * sc_ref[...]
        t = jnp.dot(a_ref[...], hv, preferred_element_type=F32)
        if has_deg:
            t = t / deg_ref[...]
        out = jnp.maximum(jnp.dot(t, wt_ref[...], preferred_element_type=F32)
                          + b_ref[...], 0.0)
        if has_resid:
            out = out + r_ref[...]
        if mask_rows:
            ri = lax.broadcasted_iota(jnp.int32, out.shape, 0)
            out = jnp.where(ri < mask_rows, out, 0.0)
        o_ref[...] = out
        if has_resid2:
            o2_ref[...] = out + r2_ref[...]
        if has_pool:
            s = jnp.sum(out * p_ref[...], axis=1, keepdims=True) + pb_ref[...]
            s = jax.nn.sigmoid(s)
            if score_kk:
                ri1 = lax.broadcasted_iota(jnp.int32, s.shape, 0)
                s = jnp.where(ri1 < score_kk, s, -1.0)
            s_ref[...] = s

    inputs = [A, h_in, Wt, b]
    if has_deg:
        inputs.append(deg)
    if has_scale:
        inputs.append(scale)
    if has_resid:
        inputs.append(resid)
    if has_resid2:
        inputs.append(resid2)
    if has_pool:
        inputs.extend(pool)
    out_shape = [jax.ShapeDtypeStruct((rows, D), F32)]
    if has_resid2:
        out_shape.append(jax.ShapeDtypeStruct((rows, D), F32))
    if has_pool:
        out_shape.append(jax.ShapeDtypeStruct((rows, 1), F32))
    res = pl.pallas_call(body, out_shape=out_shape, name=name)(*inputs)
    return res[0] if len(res) == 1 else tuple(res)


def _transpose(x, name="transpose"):
    P = x.shape[0]
    BLK = 256

    def body(x_ref, o_ref):
        o_ref[...] = x_ref[...].T

    return pl.pallas_call(
        body,
        grid=(P // BLK, P // BLK),
        in_specs=[pl.BlockSpec((BLK, BLK), lambda i, j: (i, j))],
        out_specs=pl.BlockSpec((BLK, BLK), lambda i, j: (j, i)),
        out_shape=jax.ShapeDtypeStruct((P, P), x.dtype),
        name=name,
    )(x)


def _adj_pool(Crows, Drows, kk, *, emit_t=False, name="adj_pool"):
    P, M = Crows.shape
    BLK = 128

    def body(c_ref, d_ref, b_ref, deg_ref, *rest):
        i = pl.program_id(0)
        cb = (c_ref[...] != 0).astype(BF16)
        db = (d_ref[...] != 0).astype(BF16)
        cnt = lax.dot_general(cb, db, (((1,), (1,)), ((), ())),
                              preferred_element_type=F32)
        ri = i * BLK + lax.broadcasted_iota(jnp.int32, (BLK, P), 0)
        ci = lax.broadcasted_iota(jnp.int32, (BLK, P), 1)
        Bv = jnp.where((cnt != 0) & (ri < kk) & (ci < kk), 1.0, 0.0)
        b_ref[...] = Bv
        rs = jnp.sum(Bv, axis=1, keepdims=True)
        ri1 = i * BLK + lax.broadcasted_iota(jnp.int32, (BLK, 1), 0)
        deg_ref[...] = jnp.where(ri1 < kk, rs, 1.0)
        if emit_t:
            rest[0][...] = Bv.T

    out_specs = [pl.BlockSpec((BLK, P), lambda i: (i, 0)),
                 pl.BlockSpec((BLK, 1), lambda i: (i, 0))]
    out_shape = [jax.ShapeDtypeStruct((P, P), F32),
                 jax.ShapeDtypeStruct((P, 1), F32)]
    if emit_t:
        out_specs.append(pl.BlockSpec((P, BLK), lambda i: (0, i)))
        out_shape.append(jax.ShapeDtypeStruct((P, P), F32))

    return pl.pallas_call(
        body,
        grid=(P // BLK,),
        in_specs=[pl.BlockSpec((BLK, M), lambda i: (i, 0)),
                  pl.BlockSpec((P, M), lambda i: (0, 0))],
        out_specs=out_specs,
        out_shape=out_shape,
        name=name,
    )(Crows, Drows)


def _rank(s_col, s_row, name="rank"):
    P = s_col.shape[0]
    BLK = 128

    def body(sc_ref, sr_ref, o_ref):
        i = pl.program_id(0)
        sc = sc_ref[...]
        sr = sr_ref[...]
        ri = i * BLK + lax.broadcasted_iota(jnp.int32, (BLK, P), 0)
        ci = lax.broadcasted_iota(jnp.int32, (BLK, P), 1)
        before = (sr > sc) | ((sr == sc) & (ci < ri))
        o_ref[...] = jnp.sum(before.astype(F32), axis=1, keepdims=True)

    return pl.pallas_call(
        body,
        grid=(P // BLK,),
        in_specs=[pl.BlockSpec((BLK, 1), lambda i: (i, 0)),
                  pl.BlockSpec((1, P), lambda i: (0, 0))],
        out_specs=pl.BlockSpec((BLK, 1), lambda i: (i, 0)),
        out_shape=jax.ShapeDtypeStruct((P, 1), F32),
        name=name,
    )(s_col, s_row)


def _select(rank_row, s_row, name="select"):
    P = rank_row.shape[1]
    BLK = 128

    def body(r_ref, s_ref, i_ref, v_ref):
        i = pl.program_id(0)
        rr = r_ref[...]
        sr = s_ref[...]
        rg = (i * BLK + lax.broadcasted_iota(jnp.int32, (BLK, P), 0)).astype(F32)
        match = (rr == rg).astype(F32)
        ci = lax.broadcasted_iota(jnp.int32, (BLK, P), 1).astype(F32)
        i_ref[...] = jnp.sum(match * ci, axis=1, keepdims=True).astype(jnp.int32)
        v_ref[...] = jnp.sum(match * sr, axis=1, keepdims=True)

    return pl.pallas_call(
        body,
        grid=(P // BLK,),
        in_specs=[pl.BlockSpec((1, P), lambda i: (0, 0)),
                  pl.BlockSpec((1, P), lambda i: (0, 0))],
        out_specs=[pl.BlockSpec((BLK, 1), lambda i: (i, 0)),
                   pl.BlockSpec((BLK, 1), lambda i: (i, 0))],
        out_shape=[jax.ShapeDtypeStruct((P, 1), jnp.int32),
                   jax.ShapeDtypeStruct((P, 1), F32)],
        name=name,
    )(rank_row, s_row)


def _gather_multi(tables, idx, B, CH):
    info = plsc.get_sparse_core_info()
    NW = info.num_cores * info.num_subcores
    nw_active = NW
    b_per_w = B // nw_active
    while b_per_w % 8 or B % nw_active:
        nw_active -= 1
        b_per_w = B // nw_active
    n_ch = b_per_w // CH
    mesh = plsc.VectorSubcoreMesh(core_axis_name="c", subcore_axis_name="s")

    scratch = [pltpu.VMEM((CH,), jnp.int32)]
    for t in tables:
        scratch.append(pltpu.VMEM((CH, t.shape[1]), t.dtype))
    scratch.append(pltpu.SemaphoreType.DMA)

    @functools.partial(
        pl.kernel, mesh=mesh,
        out_type=[jax.ShapeDtypeStruct((B, t.shape[1]), t.dtype)
                  for t in tables],
        scratch_types=scratch,
    )
    def k(*refs):
        nt = len(tables)
        tab = refs[:nt]
        idx_hbm = refs[nt]
        outs = refs[nt + 1:2 * nt + 1]
        idx_v = refs[2 * nt + 1]
        bufs = refs[2 * nt + 2:3 * nt + 2]
        sem = refs[3 * nt + 2]
        wid = lax.axis_index("s") * info.num_cores + lax.axis_index("c")

        @pl.when(wid < nw_active)
        def _():
            base = wid * b_per_w
            for c in range(n_ch):
                off = base + c * CH
                pltpu.sync_copy(idx_hbm.at[pl.ds(off, CH)], idx_v)
                handles = [pltpu.async_copy(tab[t].at[idx_v], bufs[t], sem)
                           for t in range(nt)]
                for hdl in handles:
                    hdl.wait()
                for t in range(nt):
                    pltpu.sync_copy(bufs[t], outs[t].at[pl.ds(off, CH)])

    return k(*tables, idx)


def kernel(g, h, params):
    p = params
    W0t = p["down_W"][0].T
    W1t = p["down_W"][1].T
    Wbt = p["bottom_W"].T
    Wu0t = p["up_W"][0].T
    Wu1t = p["up_W"][1].T
    b0 = p["down_b"][0].reshape(1, D)
    b1 = p["down_b"][1].reshape(1, D)
    bb = p["bottom_b"].reshape(1, D)
    bu0 = p["up_b"][0].reshape(1, D)
    bu1 = p["up_b"][1].reshape(1, D)
    p0 = p["pool_W"][0].reshape(1, D)
    p1 = p["pool_W"][1].reshape(1, D)
    pb0 = p["pool_b"][0].reshape(1, 1)
    pb1 = p["pool_b"][1].reshape(1, 1)

    h1, s1 = _gcn(g, h, W0t, b0, pool=(p0, pb0), name="gcn_down0")
    rank1 = _rank(s1, s1.reshape(1, N0), name="rank1")
    idxs1, vals1 = _select(rank1.reshape(1, N0), s1.reshape(1, N0),
                           name="select1")
    idx1f = idxs1.reshape(N0)
    rank1i = rank1.astype(jnp.int32).reshape(N0)

    gT = _transpose(g, name="transpose_g")
    G_r, GT_r, h1_r = _gather_multi([g, gT, h1], idx1f, P1, 16)

    B1, deg1, B1T = _adj_pool(G_r, GT_r, KK1, emit_t=True, name="adj_pool1")

    h2, s2 = _gcn(B1, h1_r, W1t, b1, deg=deg1, scale=vals1[:P1],
                  pool=(p1, pb1), score_kk=KK1, name="gcn_down1")
    rank2 = _rank(s2, s2.reshape(1, P1), name="rank2")
    idxs2, vals2 = _select(rank2.reshape(1, P1), s2.reshape(1, P1),
                           name="select2")
    idx2f = idxs2.reshape(P1)
    rank2i = rank2.astype(jnp.int32).reshape(P1)

    B1_r, B1T_r, h2_r = _gather_multi([B1, B1T, h2], idx2f, P2, 16)

    B2, deg2 = _adj_pool(B1_r, B1T_r, KK2, name="adj_pool2")

    hb = _gcn(B2, h2_r, Wbt, bb, deg=deg2, scale=vals2[:P2], mask_rows=KK2,
              name="gcn_bottom")

    hb_pad = jnp.pad(hb, ((0, P1 - P2), (0, 0)))
    (nh_b,) = _gather_multi([hb_pad], rank2i, P1, 64)
    h_u0 = _gcn(B1, nh_b, Wu0t, bu0, deg=deg1, resid=h2, mask_rows=KK1,
                name="gcn_up0")

    h_u0_pad = jnp.pad(h_u0, ((0, N0 - P1), (0, 0)))
    (nh,) = _gather_multi([h_u0_pad], rank1i, N0, 64)
    h_u1, h_fin = _gcn(g, nh, Wu1t, bu1, resid=h1, resid2=h, name="gcn_up1")

    return (h_u0[:KK1], h_u1, h_fin)
